# baseline (speedup 1.0000x reference)
import functools

import jax
import jax.numpy as jnp
from jax import lax
from jax.experimental import pallas as pl
from jax.experimental.pallas import tpu as pltpu
from jax.experimental.pallas import tpu_sc as plsc

_NUM_CODES = 1024
_DIM = 64
_ROWS = 8192
_LEAD = 8
_BLOCK = 512
_COMMIT = 0.25

_NW = 32
_BPW = _ROWS // _NW
_GRP = _BPW // 16


def _tc_body(x_ref, emb_ref, s2_ref, iota_ref, idx_ref, msum_ref):
    x = x_ref[...].reshape(_BLOCK, _DIM)
    mm2 = jnp.dot(x + x, emb_ref[...], preferred_element_type=jnp.float32)
    s1 = jnp.sum(x * x, axis=1, keepdims=True)
    d = (s1 + s2_ref[...]) - mm2
    m = jnp.min(d, axis=1, keepdims=True)
    idxf = jnp.min(jnp.where(d == m, iota_ref[...], float(_NUM_CODES)), axis=1)
    idx_ref[...] = idxf.astype(jnp.int32)
    part = jnp.sum(m).reshape(1, 1)

    @pl.when(pl.program_id(0) == 0)
    def _():
        msum_ref[...] = part

    @pl.when(pl.program_id(0) > 0)
    def _():
        msum_ref[...] = msum_ref[...] + part


_tc_call = pl.pallas_call(
    _tc_body,
    grid=(_ROWS // _BLOCK,),
    in_specs=[
        pl.BlockSpec((1, _BLOCK, _DIM), lambda i: (i // 2, i % 2, 0)),
        pl.BlockSpec((_DIM, _NUM_CODES), lambda i: (0, 0)),
        pl.BlockSpec((1, _NUM_CODES), lambda i: (0, 0)),
        pl.BlockSpec((1, _NUM_CODES), lambda i: (0, 0)),
    ],
    out_specs=[
        pl.BlockSpec((_BLOCK,), lambda i: (i,)),
        pl.BlockSpec((1, 1), lambda i: (0, 0)),
    ],
    out_shape=[
        jax.ShapeDtypeStruct((_ROWS,), jnp.int32),
        jax.ShapeDtypeStruct((1, 1), jnp.float32),
    ],
)


@functools.cache
def _sc_gather_call():
    @functools.partial(
        pl.kernel,
        mesh=plsc.VectorSubcoreMesh(core_axis_name="c", subcore_axis_name="s"),
        out_type=jax.ShapeDtypeStruct((_LEAD, 1024, _DIM), jnp.float32),
        scratch_types=[
            pltpu.VMEM_SHARED((_DIM * _NUM_CODES,), jnp.float32),
            pltpu.VMEM((_DIM * _NUM_CODES,), jnp.float32),
            pltpu.VMEM((_BPW,), jnp.int32),
            pltpu.VMEM((_BPW, _DIM), jnp.float32),
            pltpu.SemaphoreType.DMA,
        ],
        compiler_params=pltpu.CompilerParams(needs_layout_passes=False),
    )
    def _sc_gather(
        table_hbm, idx_hbm, out_hbm, table_s, table_v, idx_v, rows_v, sem
    ):
        sid = lax.axis_index("s")
        wid = sid * 2 + lax.axis_index("c")

        with jax.named_scope("sc_stage"):
            @pl.when(sid == 0)
            def _():
                pltpu.sync_copy(table_hbm, table_s)

            pltpu.sync_copy(idx_hbm.at[pl.ds(wid * _BPW, _BPW)], idx_v)
            plsc.subcore_barrier()
            pltpu.sync_copy(table_s, table_v)
        lanes = lax.iota(jnp.int32, 16)

        with jax.named_scope("sc_gather"):
            @pl.loop(0, _GRP)
            def _(g):
                ridx = idx_v[pl.ds(g * 16, 16)]
                rv = lanes + g * 16
                gidx = ridx
                for k in range(_DIM):
                    vals = plsc.load_gather(table_v, [gidx])
                    plsc.store_scatter(
                        rows_v, [rv, jnp.full((16,), k, jnp.int32)], vals
                    )
                    if k + 1 < _DIM:
                        gidx = gidx + _NUM_CODES

        with jax.named_scope("sc_writeback"):
            pltpu.sync_copy(
                rows_v, out_hbm.at[wid // 4, pl.ds((wid % 4) * _BPW, _BPW)]
            )

    return _sc_gather


def kernel(inputs, embeddings):
    s2 = jnp.sum(embeddings ** 2, axis=0)[None, :]
    iota_f = jnp.arange(_NUM_CODES, dtype=jnp.float32)[None, :]
    idx_flat, msum = _tc_call(inputs, embeddings, s2, iota_f)
    quant = _sc_gather_call()(embeddings.reshape(-1), idx_flat)
    loss = _COMMIT * (msum[0, 0] / (_ROWS * _DIM))
    return loss, quant, idx_flat.reshape(inputs.shape[:-1])


# speedup vs baseline: 1.0259x; 1.0259x over previous
import functools

import jax
import jax.numpy as jnp
from jax import lax
from jax.experimental import pallas as pl
from jax.experimental.pallas import tpu as pltpu
from jax.experimental.pallas import tpu_sc as plsc

_NUM_CODES = 1024
_DIM = 64
_ROWS = 8192
_LEAD = 8
_BLOCK = 512
_COMMIT = 0.25

_NW = 32
_BPW = _ROWS // _NW
_GRP = _BPW // 16


def _tc_body(x_ref, emb_ref, s2_ref, iota_ref, idx_ref, msum_ref):
    x = x_ref[...].reshape(_BLOCK, _DIM)
    mm2 = jnp.dot(x + x, emb_ref[...], preferred_element_type=jnp.float32)
    s1 = jnp.sum(x * x, axis=1, keepdims=True)
    d = (s1 + s2_ref[...]) - mm2
    m = jnp.min(d, axis=1, keepdims=True)
    idxf = jnp.min(jnp.where(d == m, iota_ref[...], float(_NUM_CODES)), axis=1)
    idx_ref[...] = idxf.astype(jnp.int32)
    part = jnp.sum(m).reshape(1, 1)

    @pl.when(pl.program_id(0) == 0)
    def _():
        msum_ref[...] = part

    @pl.when(pl.program_id(0) > 0)
    def _():
        msum_ref[...] = msum_ref[...] + part


_tc_call = pl.pallas_call(
    _tc_body,
    grid=(_ROWS // _BLOCK,),
    in_specs=[
        pl.BlockSpec((1, _BLOCK, _DIM), lambda i: (i // 2, i % 2, 0)),
        pl.BlockSpec((_DIM, _NUM_CODES), lambda i: (0, 0)),
        pl.BlockSpec((1, _NUM_CODES), lambda i: (0, 0)),
        pl.BlockSpec((1, _NUM_CODES), lambda i: (0, 0)),
    ],
    out_specs=[
        pl.BlockSpec((_BLOCK,), lambda i: (i,)),
        pl.BlockSpec((1, 1), lambda i: (0, 0)),
    ],
    out_shape=[
        jax.ShapeDtypeStruct((_ROWS,), jnp.int32),
        jax.ShapeDtypeStruct((1, 1), jnp.float32),
    ],
)


@functools.cache
def _sc_gather_call():
    @functools.partial(
        pl.kernel,
        mesh=plsc.VectorSubcoreMesh(core_axis_name="c", subcore_axis_name="s"),
        out_type=jax.ShapeDtypeStruct((_LEAD, 1024, _DIM), jnp.float32),
        scratch_types=[
            pltpu.VMEM_SHARED((_DIM, _NUM_CODES), jnp.float32),
            pltpu.VMEM((_DIM, _NUM_CODES), jnp.float32),
            pltpu.VMEM((_BPW,), jnp.int32),
            pltpu.VMEM((_BPW, _DIM), jnp.float32),
            pltpu.SemaphoreType.DMA,
        ],
        compiler_params=pltpu.CompilerParams(needs_layout_passes=False),
    )
    def _sc_gather(
        table_hbm, idx_hbm, out_hbm, table_s, table_v, idx_v, rows_v, sem
    ):
        sid = lax.axis_index("s")
        wid = sid * 2 + lax.axis_index("c")

        with jax.named_scope("sc_stage"):
            @pl.when(sid == 0)
            def _():
                pltpu.sync_copy(table_hbm, table_s)

            pltpu.sync_copy(idx_hbm.at[pl.ds(wid * _BPW, _BPW)], idx_v)
            plsc.subcore_barrier()
            pltpu.sync_copy(table_s, table_v)
        lanes = lax.iota(jnp.int32, 16)

        with jax.named_scope("sc_gather"):
            @pl.loop(0, _GRP)
            def _(g):
                ridx = idx_v[pl.ds(g * 16, 16)]
                rv = lanes + g * 16

                @pl.loop(0, _DIM)
                def _(k):
                    kf = jnp.full((16,), k, jnp.int32)
                    vals = plsc.load_gather(table_v, [kf, ridx])
                    plsc.store_scatter(rows_v, [rv, kf], vals)

        with jax.named_scope("sc_writeback"):
            pltpu.sync_copy(
                rows_v, out_hbm.at[wid // 4, pl.ds((wid % 4) * _BPW, _BPW)]
            )

    return _sc_gather


def kernel(inputs, embeddings):
    s2 = jnp.sum(embeddings ** 2, axis=0)[None, :]
    iota_f = jnp.arange(_NUM_CODES, dtype=jnp.float32)[None, :]
    idx_flat, msum = _tc_call(inputs, embeddings, s2, iota_f)
    quant = _sc_gather_call()(embeddings, idx_flat)
    loss = _COMMIT * (msum[0, 0] / (_ROWS * _DIM))
    return loss, quant, idx_flat.reshape(inputs.shape[:-1])


# speedup vs baseline: 1.1267x; 1.0983x over previous
import functools

import jax
import jax.numpy as jnp
from jax import lax
from jax.experimental import pallas as pl
from jax.experimental.pallas import tpu as pltpu
from jax.experimental.pallas import tpu_sc as plsc

_NUM_CODES = 1024
_DIM = 64
_ROWS = 8192
_LEAD = 8
_BLOCK = 512
_COMMIT = 0.25

_NW = 32
_BPW = _ROWS // _NW
_GRP = _BPW // 16


def _tc_body(x_ref, emb_ref, s2_ref, iota_ref, idx_ref, msum_ref):
    x = x_ref[...].reshape(_BLOCK, _DIM)
    mm2 = jnp.dot(x + x, emb_ref[...], preferred_element_type=jnp.float32)
    s1 = jnp.sum(x * x, axis=1, keepdims=True)
    d = (s1 + s2_ref[...]) - mm2
    m = jnp.min(d, axis=1, keepdims=True)
    idxf = jnp.min(jnp.where(d == m, iota_ref[...], float(_NUM_CODES)), axis=1)
    idx_ref[...] = idxf.astype(jnp.int32)
    part = jnp.sum(m).reshape(1, 1)

    @pl.when(pl.program_id(0) == 0)
    def _():
        msum_ref[...] = part

    @pl.when(pl.program_id(0) > 0)
    def _():
        msum_ref[...] = msum_ref[...] + part


_tc_call = pl.pallas_call(
    _tc_body,
    grid=(_ROWS // _BLOCK,),
    in_specs=[
        pl.BlockSpec((1, _BLOCK, _DIM), lambda i: (i // 2, i % 2, 0)),
        pl.BlockSpec((_DIM, _NUM_CODES), lambda i: (0, 0)),
        pl.BlockSpec((1, _NUM_CODES), lambda i: (0, 0)),
        pl.BlockSpec((1, _NUM_CODES), lambda i: (0, 0)),
    ],
    out_specs=[
        pl.BlockSpec((_BLOCK,), lambda i: (i,)),
        pl.BlockSpec((1, 1), lambda i: (0, 0)),
    ],
    out_shape=[
        jax.ShapeDtypeStruct((_ROWS,), jnp.int32),
        jax.ShapeDtypeStruct((1, 1), jnp.float32),
    ],
)


@functools.cache
def _sc_gather_call():
    @functools.partial(
        pl.kernel,
        mesh=plsc.VectorSubcoreMesh(core_axis_name="c", subcore_axis_name="s"),
        out_type=jax.ShapeDtypeStruct((_LEAD, 1024, _DIM), jnp.float32),
        scratch_types=[
            pltpu.VMEM_SHARED((_NUM_CODES, _DIM), jnp.float32),
            pltpu.VMEM((_NUM_CODES, _DIM), jnp.float32),
            pltpu.VMEM((_BPW,), jnp.int32),
            pltpu.VMEM((_BPW, _DIM), jnp.float32),
            pltpu.SemaphoreType.DMA,
        ],
        compiler_params=pltpu.CompilerParams(
            needs_layout_passes=False, use_tc_tiling_on_sc=False
        ),
    )
    def _sc_gather(
        table_hbm, idx_hbm, out_hbm, table_s, table_v, idx_v, rows_v, sem
    ):
        sid = lax.axis_index("s")
        wid = sid * 2 + lax.axis_index("c")

        with jax.named_scope("sc_stage"):
            @pl.when(sid == 0)
            def _():
                pltpu.sync_copy(table_hbm, table_s)

            pltpu.sync_copy(idx_hbm.at[pl.ds(wid * _BPW, _BPW)], idx_v)
            plsc.subcore_barrier()
            pltpu.sync_copy(table_s, table_v)
        lanes = lax.iota(jnp.int32, 16)

        with jax.named_scope("sc_gather"):
            @pl.loop(0, _GRP)
            def _(g):
                ridx = idx_v[pl.ds(g * 16, 16)]
                for j in range(16):
                    cf = lax.gather(
                        ridx,
                        jnp.full((16, 1), j, jnp.int32),
                        lax.GatherDimensionNumbers(
                            offset_dims=(),
                            collapsed_slice_dims=(0,),
                            start_index_map=(0,),
                        ),
                        slice_sizes=(1,),
                        mode=lax.GatherScatterMode.PROMISE_IN_BOUNDS,
                    )
                    r = g * 16 + j
                    for m in range(_DIM // 16):
                        vals = plsc.load_gather(table_v, [cf, lanes + 16 * m])
                        rows_v[r, pl.ds(16 * m, 16)] = vals

        with jax.named_scope("sc_writeback"):
            pltpu.sync_copy(
                rows_v, out_hbm.at[wid // 4, pl.ds((wid % 4) * _BPW, _BPW)]
            )

    return _sc_gather


def kernel(inputs, embeddings):
    s2 = jnp.sum(embeddings ** 2, axis=0)[None, :]
    iota_f = jnp.arange(_NUM_CODES, dtype=jnp.float32)[None, :]
    idx_flat, msum = _tc_call(inputs, embeddings, s2, iota_f)
    quant = _sc_gather_call()(embeddings.T, idx_flat)
    loss = _COMMIT * (msum[0, 0] / (_ROWS * _DIM))
    return loss, quant, idx_flat.reshape(inputs.shape[:-1])


# speedup vs baseline: 1.1598x; 1.0293x over previous
import functools

import jax
import jax.numpy as jnp
from jax import lax
from jax.experimental import pallas as pl
from jax.experimental.pallas import tpu as pltpu
from jax.experimental.pallas import tpu_sc as plsc

_NUM_CODES = 1024
_DIM = 64
_ROWS = 8192
_LEAD = 8
_BLOCK = 512
_COMMIT = 0.25

_NW = 32
_BPW = _ROWS // _NW
_GRP = _BPW // 16


def _tc_body(x_ref, emb_ref, s2_ref, iota_ref, idx_ref, msum_ref):
    x = x_ref[...].reshape(_BLOCK, _DIM)
    mm2 = jnp.dot(x + x, emb_ref[...], preferred_element_type=jnp.float32)
    s1 = jnp.sum(x * x, axis=1, keepdims=True)
    d = (s1 + s2_ref[...]) - mm2
    m = jnp.min(d, axis=1, keepdims=True)
    idxf = jnp.min(jnp.where(d == m, iota_ref[...], float(_NUM_CODES)), axis=1)
    idx_ref[...] = idxf.astype(jnp.int32)
    part = jnp.sum(m).reshape(1, 1)

    @pl.when(pl.program_id(0) == 0)
    def _():
        msum_ref[...] = part

    @pl.when(pl.program_id(0) > 0)
    def _():
        msum_ref[...] = msum_ref[...] + part


_tc_call = pl.pallas_call(
    _tc_body,
    grid=(_ROWS // _BLOCK,),
    in_specs=[
        pl.BlockSpec((1, _BLOCK, _DIM), lambda i: (i // 2, i % 2, 0)),
        pl.BlockSpec((_DIM, _NUM_CODES), lambda i: (0, 0)),
        pl.BlockSpec((1, _NUM_CODES), lambda i: (0, 0)),
        pl.BlockSpec((1, _NUM_CODES), lambda i: (0, 0)),
    ],
    out_specs=[
        pl.BlockSpec((_BLOCK,), lambda i: (i,)),
        pl.BlockSpec((1, 1), lambda i: (0, 0)),
    ],
    out_shape=[
        jax.ShapeDtypeStruct((_ROWS,), jnp.int32),
        jax.ShapeDtypeStruct((1, 1), jnp.float32),
    ],
)


@functools.cache
def _sc_gather_call():
    @functools.partial(
        pl.kernel,
        mesh=plsc.VectorSubcoreMesh(core_axis_name="c", subcore_axis_name="s"),
        out_type=jax.ShapeDtypeStruct((_LEAD, _DIM, 1024), jnp.float32),
        scratch_types=[
            pltpu.VMEM_SHARED((_DIM, _NUM_CODES), jnp.float32),
            pltpu.VMEM((_DIM, _NUM_CODES), jnp.float32),
            pltpu.VMEM((_BPW,), jnp.int32),
            pltpu.VMEM((_DIM, _BPW), jnp.float32),
            pltpu.SemaphoreType.DMA,
        ],
        compiler_params=pltpu.CompilerParams(
            needs_layout_passes=False, use_tc_tiling_on_sc=False
        ),
    )
    def _sc_gather(
        table_hbm, idx_hbm, out_hbm, table_s, table_v, idx_v, rows_v, sem
    ):
        sid = lax.axis_index("s")
        wid = sid * 2 + lax.axis_index("c")

        with jax.named_scope("sc_stage"):
            @pl.when(sid == 0)
            def _():
                pltpu.sync_copy(table_hbm, table_s)

            pltpu.sync_copy(idx_hbm.at[pl.ds(wid * _BPW, _BPW)], idx_v)
            plsc.subcore_barrier()
            pltpu.sync_copy(table_s, table_v)
        lanes = lax.iota(jnp.int32, 16)

        with jax.named_scope("sc_gather"):
            @pl.loop(0, _GRP)
            def _(g):
                ridx = idx_v[pl.ds(g * 16, 16)]
                rv = lanes + g * 16
                for k in range(_DIM):
                    kf = jnp.full((16,), k, jnp.int32)
                    vals = plsc.load_gather(table_v, [kf, ridx])
                    plsc.store_scatter(rows_v, [kf, rv], vals)

        with jax.named_scope("sc_writeback"):
            pltpu.sync_copy(
                rows_v,
                out_hbm.at[wid // 4, :, pl.ds((wid % 4) * _BPW, _BPW)],
            )

    return _sc_gather


def kernel(inputs, embeddings):
    s2 = jnp.sum(embeddings ** 2, axis=0)[None, :]
    iota_f = jnp.arange(_NUM_CODES, dtype=jnp.float32)[None, :]
    idx_flat, msum = _tc_call(inputs, embeddings, s2, iota_f)
    quant_t = _sc_gather_call()(embeddings, idx_flat)
    loss = _COMMIT * (msum[0, 0] / (_ROWS * _DIM))
    return (
        loss,
        quant_t.transpose(0, 2, 1),
        idx_flat.reshape(inputs.shape[:-1]),
    )


# speedup vs baseline: 1.1641x; 1.0037x over previous
import functools

import jax
import jax.numpy as jnp
from jax import lax
from jax.experimental import pallas as pl
from jax.experimental.pallas import tpu as pltpu
from jax.experimental.pallas import tpu_sc as plsc

_NUM_CODES = 1024
_DIM = 64
_ROWS = 8192
_LEAD = 8
_BLOCK = 1024
_COMMIT = 0.25

_NW = 32
_BPW = _ROWS // _NW
_GRP = _BPW // 16


def _tc_body(x_ref, emb_ref, s2_ref, iota_ref, idx_ref, msum_ref):
    x = x_ref[...].reshape(_BLOCK, _DIM)
    mm2 = jnp.dot(x + x, emb_ref[...], preferred_element_type=jnp.float32)
    s1 = jnp.sum(x * x, axis=1, keepdims=True)
    d = (s1 + s2_ref[...]) - mm2
    m = jnp.min(d, axis=1, keepdims=True)
    idxf = jnp.min(jnp.where(d == m, iota_ref[...], float(_NUM_CODES)), axis=1)
    idx_ref[...] = idxf.astype(jnp.int32)
    part = jnp.sum(m).reshape(1, 1)

    @pl.when(pl.program_id(0) == 0)
    def _():
        msum_ref[...] = part

    @pl.when(pl.program_id(0) > 0)
    def _():
        msum_ref[...] = msum_ref[...] + part


_tc_call = pl.pallas_call(
    _tc_body,
    grid=(_ROWS // _BLOCK,),
    in_specs=[
        pl.BlockSpec((1, _BLOCK, _DIM), lambda i: (i, 0, 0)),
        pl.BlockSpec((_DIM, _NUM_CODES), lambda i: (0, 0)),
        pl.BlockSpec((1, _NUM_CODES), lambda i: (0, 0)),
        pl.BlockSpec((1, _NUM_CODES), lambda i: (0, 0)),
    ],
    out_specs=[
        pl.BlockSpec((_BLOCK,), lambda i: (i,)),
        pl.BlockSpec((1, 1), lambda i: (0, 0)),
    ],
    out_shape=[
        jax.ShapeDtypeStruct((_ROWS,), jnp.int32),
        jax.ShapeDtypeStruct((1, 1), jnp.float32),
    ],
)


@functools.cache
def _sc_gather_call():
    @functools.partial(
        pl.kernel,
        mesh=plsc.VectorSubcoreMesh(core_axis_name="c", subcore_axis_name="s"),
        out_type=jax.ShapeDtypeStruct((_LEAD, _DIM, 1024), jnp.float32),
        scratch_types=[
            pltpu.VMEM_SHARED((_DIM, _NUM_CODES), jnp.float32),
            pltpu.VMEM((_DIM, _NUM_CODES), jnp.float32),
            pltpu.VMEM((_BPW,), jnp.int32),
            pltpu.VMEM((_DIM, _BPW), jnp.float32),
            pltpu.SemaphoreType.DMA,
        ],
        compiler_params=pltpu.CompilerParams(
            needs_layout_passes=False, use_tc_tiling_on_sc=False
        ),
    )
    def _sc_gather(
        table_hbm, idx_hbm, out_hbm, table_s, table_v, idx_v, rows_v, sem
    ):
        sid = lax.axis_index("s")
        wid = sid * 2 + lax.axis_index("c")

        with jax.named_scope("sc_stage"):
            @pl.when(sid == 0)
            def _():
                pltpu.sync_copy(table_hbm, table_s)

            pltpu.sync_copy(idx_hbm.at[pl.ds(wid * _BPW, _BPW)], idx_v)
            plsc.subcore_barrier()
            pltpu.sync_copy(table_s, table_v)
        lanes = lax.iota(jnp.int32, 16)

        with jax.named_scope("sc_gather"):
            @pl.loop(0, _GRP)
            def _(g):
                ridx = idx_v[pl.ds(g * 16, 16)]
                rv = lanes + g * 16
                for k in range(_DIM):
                    kf = jnp.full((16,), k, jnp.int32)
                    vals = plsc.load_gather(table_v, [kf, ridx])
                    plsc.store_scatter(rows_v, [kf, rv], vals)

        with jax.named_scope("sc_writeback"):
            pltpu.sync_copy(
                rows_v,
                out_hbm.at[wid // 4, :, pl.ds((wid % 4) * _BPW, _BPW)],
            )

    return _sc_gather


def kernel(inputs, embeddings):
    s2 = jnp.sum(embeddings ** 2, axis=0)[None, :]
    iota_f = jnp.arange(_NUM_CODES, dtype=jnp.float32)[None, :]
    idx_flat, msum = _tc_call(inputs, embeddings, s2, iota_f)
    quant_t = _sc_gather_call()(embeddings, idx_flat)
    loss = _COMMIT * (msum[0, 0] / (_ROWS * _DIM))
    return (
        loss,
        quant_t.transpose(0, 2, 1),
        idx_flat.reshape(inputs.shape[:-1]),
    )
